# Initial kernel scaffold; baseline (speedup 1.0000x reference)
import jax
import jax.numpy as jnp
from jax.experimental import pallas as pl


def kernel(x, table):
    raise NotImplementedError("write your pallas kernel here")


# speedup vs baseline: 1.2432x; 1.2432x over previous
import functools

import jax
import jax.numpy as jnp
from jax import lax
from jax.experimental import pallas as pl
from jax.experimental.pallas import tpu as pltpu
from jax.experimental.pallas import tpu_sc as plsc

NC = 2
NS = 16
NW = NC * NS
LANE = 16


@functools.partial(jax.jit, static_argnames=("bb", "ll", "dd", "dp", "g"))
def _emb_mean(xi, table_p, bb, ll, dd, dp, g):
    tok_w = (bb // NW) * ll
    nchunk = (bb // NW) // g
    gl = g * ll
    pieces = []
    off = 0
    while off < gl:
        n = min(128, gl - off)
        if gl - off > 128 and n % 8 != 0:
            n -= n % 8
        pieces.append((off, n))
        off += n
    offs = list(range(0, dp - LANE + 1, LANE))
    mesh = plsc.VectorSubcoreMesh(
        core_axis_name="c", subcore_axis_name="s", num_cores=NC, num_subcores=NS
    )

    @functools.partial(
        pl.kernel,
        out_type=jax.ShapeDtypeStruct((bb * dd,), jnp.float32),
        mesh=mesh,
        scratch_types=[
            pltpu.VMEM((tok_w,), jnp.int32),
            pltpu.VMEM((gl, dp), jnp.float32),
            pltpu.VMEM((g * dd + LANE,), jnp.float32),
            pltpu.SemaphoreType.DMA,
        ],
        compiler_params=pltpu.CompilerParams(use_tc_tiling_on_sc=False),
    )
    def k(x_hbm, tab_hbm, out_hbm, idx_v, rows_v, out_v, sem):
        w = lax.axis_index("s") * NC + lax.axis_index("c")
        pltpu.sync_copy(x_hbm.at[pl.ds(w * tok_w, tok_w)], idx_v)
        inv_l = jnp.float32(1.0 / ll)

        @pl.loop(0, nchunk)
        def _(c):
            base = c * gl
            cps = [
                pltpu.async_copy(
                    tab_hbm.at[idx_v.at[pl.ds(base + o, n)]],
                    rows_v.at[pl.ds(o, n)],
                    sem,
                )
                for o, n in pieces
            ]
            for cp in cps:
                cp.wait()
            for r in range(g):
                def body(l, acc):
                    t = r * ll + l
                    return tuple(
                        acc[i] + rows_v[t, pl.ds(o, LANE)]
                        for i, o in enumerate(offs)
                    )
                acc0 = tuple(jnp.zeros((LANE,), jnp.float32) for _ in offs)
                acc = lax.fori_loop(0, ll, body, acc0)
                for i, o in enumerate(offs):
                    out_v[pl.ds(r * dd + o, LANE)] = acc[i] * inv_l
            pltpu.sync_copy(
                out_v.at[pl.ds(0, g * dd)],
                out_hbm.at[pl.ds((w * nchunk + c) * g * dd, g * dd)],
            )

    return k(xi, table_p)


def kernel(x, table):
    bb, ll = x.shape
    _, dd = table.shape
    dp = (dd + LANE - 1) // LANE * LANE
    xi = x.reshape(-1).astype(jnp.int32)
    table_p = jnp.pad(table, ((0, 0), (0, dp - dd)))
    out = _emb_mean(xi, table_p, bb, ll, dd, dp, 4)
    return out.reshape(bb, dd)


# speedup vs baseline: 1.9844x; 1.5962x over previous
import functools

import jax
import jax.numpy as jnp
from jax import lax
from jax.experimental import pallas as pl
from jax.experimental.pallas import tpu as pltpu
from jax.experimental.pallas import tpu_sc as plsc

NC = 2
NS = 16
NW = NC * NS
LANE = 16


@functools.partial(jax.jit, static_argnames=("bb", "ll", "dd", "dp", "g"))
def _emb_mean(xi, table_p, bb, ll, dd, dp, g):
    tok_w = (bb // NW) * ll
    nchunk = (bb // NW) // g
    gl = g * ll
    pieces = []
    off = 0
    while off < gl:
        n = min(128, gl - off)
        if gl - off > 128 and n % 8 != 0:
            n -= n % 8
        pieces.append((off, n))
        off += n
    offs = list(range(0, dp - LANE + 1, LANE))
    mesh = plsc.VectorSubcoreMesh(
        core_axis_name="c", subcore_axis_name="s", num_cores=NC, num_subcores=NS
    )

    @functools.partial(
        pl.kernel,
        out_type=jax.ShapeDtypeStruct((bb * dd,), jnp.float32),
        mesh=mesh,
        scratch_types=[
            pltpu.VMEM((tok_w,), jnp.int32),
            pltpu.VMEM((gl, dp), jnp.float32),
            pltpu.VMEM((g * dd + LANE,), jnp.float32),
            pltpu.SemaphoreType.DMA,
        ],
        compiler_params=pltpu.CompilerParams(use_tc_tiling_on_sc=False),
    )
    def k(x_hbm, tab_hbm, out_hbm, idx_v, rows_v, out_v, sem):
        w = lax.axis_index("s") * NC + lax.axis_index("c")
        pltpu.sync_copy(x_hbm.at[pl.ds(w * tok_w, tok_w)], idx_v)
        inv_l = jnp.float32(1.0 / ll)

        @pl.loop(0, nchunk)
        def _(c):
            base = c * gl
            cps = [
                pltpu.async_copy(
                    tab_hbm.at[idx_v.at[pl.ds(base + o, n)]],
                    rows_v.at[pl.ds(o, n)],
                    sem,
                )
                for o, n in pieces
            ]
            for cp in cps:
                cp.wait()
            for r in range(g):
                def body(l, acc):
                    t = r * ll + l
                    return tuple(
                        acc[i] + rows_v[t, pl.ds(o, LANE)]
                        for i, o in enumerate(offs)
                    )
                acc0 = tuple(jnp.zeros((LANE,), jnp.float32) for _ in offs)
                acc = lax.fori_loop(0, ll, body, acc0)
                for i, o in enumerate(offs):
                    out_v[pl.ds(r * dd + o, LANE)] = acc[i] * inv_l
            pltpu.sync_copy(
                out_v.at[pl.ds(0, g * dd)],
                out_hbm.at[pl.ds((w * nchunk + c) * g * dd, g * dd)],
            )

    return k(xi, table_p)


def _pad_table(table, dp):
    vv, dd = table.shape
    bs = 2000
    assert vv % bs == 0

    def body(x_ref, o_ref):
        o_ref[:, :dd] = x_ref[...]
        o_ref[:, dd:] = jnp.zeros((bs, dp - dd), jnp.float32)

    return pl.pallas_call(
        body,
        grid=(vv // bs,),
        in_specs=[pl.BlockSpec((bs, dd), lambda i: (i, 0))],
        out_specs=pl.BlockSpec((bs, dp), lambda i: (i, 0)),
        out_shape=jax.ShapeDtypeStruct((vv, dp), jnp.float32),
    )(table)


def kernel(x, table):
    bb, ll = x.shape
    _, dd = table.shape
    dp = (dd + LANE - 1) // LANE * LANE
    xi = x.reshape(-1).astype(jnp.int32)
    table_p = _pad_table(table, dp)
    out = _emb_mean(xi, table_p, bb, ll, dd, dp, 4)
    return out.reshape(bb, dd)


# speedup vs baseline: 2.6392x; 1.3300x over previous
import functools

import jax
import jax.numpy as jnp
from jax import lax
from jax.experimental import pallas as pl
from jax.experimental.pallas import tpu as pltpu
from jax.experimental.pallas import tpu_sc as plsc

NC = 2
NS = 16
NW = NC * NS
LANE = 16


@functools.partial(jax.jit, static_argnames=("bb", "ll", "dd", "dp", "g"))
def _emb_mean(xi, table_p, bb, ll, dd, dp, g):
    tok_w = (bb // NW) * ll
    nchunk = (bb // NW) // g
    gl = g * ll
    pieces = []
    off = 0
    while off < gl:
        n = min(128, gl - off)
        if gl - off > 128 and n % 8 != 0:
            n -= n % 8
        pieces.append((off, n))
        off += n
    dcov = (dd + LANE - 1) // LANE * LANE
    offs = list(range(0, dcov, LANE))
    mesh = plsc.VectorSubcoreMesh(
        core_axis_name="c", subcore_axis_name="s", num_cores=NC, num_subcores=NS
    )

    @functools.partial(
        pl.kernel,
        out_type=jax.ShapeDtypeStruct((bb * dd,), jnp.float32),
        mesh=mesh,
        scratch_types=[
            pltpu.VMEM((tok_w,), jnp.int32),
            pltpu.VMEM((gl, dp), jnp.float32),
            pltpu.VMEM((g * dd + LANE,), jnp.float32),
            pltpu.SemaphoreType.DMA,
        ],
    )
    def k(x_hbm, tab_hbm, out_hbm, idx_v, rows_v, out_v, sem):
        w = lax.axis_index("s") * NC + lax.axis_index("c")
        pltpu.sync_copy(x_hbm.at[pl.ds(w * tok_w, tok_w)], idx_v)
        inv_l = jnp.float32(1.0 / ll)

        @pl.loop(0, nchunk)
        def _(c):
            base = c * gl
            cps = [
                pltpu.async_copy(
                    tab_hbm.at[idx_v.at[pl.ds(base + o, n)]],
                    rows_v.at[pl.ds(o, n)],
                    sem,
                )
                for o, n in pieces
            ]
            for cp in cps:
                cp.wait()
            for r in range(g):
                def body(l, acc):
                    t = r * ll + l
                    return tuple(
                        acc[i] + rows_v[t, pl.ds(o, LANE)]
                        for i, o in enumerate(offs)
                    )
                acc0 = tuple(jnp.zeros((LANE,), jnp.float32) for _ in offs)
                acc = lax.fori_loop(0, ll, body, acc0)
                for i, o in enumerate(offs):
                    out_v[pl.ds(r * dd + o, LANE)] = acc[i] * inv_l
            pltpu.sync_copy(
                out_v.at[pl.ds(0, g * dd)],
                out_hbm.at[pl.ds((w * nchunk + c) * g * dd, g * dd)],
            )

    return k(xi, table_p)


def _pad_table(table, dp):
    vv, dd = table.shape
    bs = 2000
    assert vv % bs == 0

    def body(x_ref, o_ref):
        o_ref[:, :dd] = x_ref[...]
        o_ref[:, dd:] = jnp.zeros((bs, dp - dd), jnp.float32)

    return pl.pallas_call(
        body,
        grid=(vv // bs,),
        in_specs=[pl.BlockSpec((bs, dd), lambda i: (i, 0))],
        out_specs=pl.BlockSpec((bs, dp), lambda i: (i, 0)),
        out_shape=jax.ShapeDtypeStruct((vv, dp), jnp.float32),
    )(table)


def kernel(x, table):
    bb, ll = x.shape
    _, dd = table.shape
    dp = (dd + 127) // 128 * 128
    xi = x.reshape(-1).astype(jnp.int32)
    table_p = _pad_table(table, dp)
    out = _emb_mean(xi, table_p, bb, ll, dd, dp, 4)
    return out.reshape(bb, dd)


# speedup vs baseline: 2.6495x; 1.0039x over previous
import functools

import jax
import jax.numpy as jnp
from jax import lax
from jax.experimental import pallas as pl
from jax.experimental.pallas import tpu as pltpu
from jax.experimental.pallas import tpu_sc as plsc

NC = 2
NS = 16
NW = NC * NS
LANE = 16


@functools.partial(jax.jit, static_argnames=("bb", "ll", "dd", "dp", "g"))
def _emb_mean(xi, table_p, bb, ll, dd, dp, g):
    tok_w = (bb // NW) * ll
    nchunk = (bb // NW) // g
    gl = g * ll
    pieces = []
    off = 0
    while off < gl:
        n = min(128, gl - off)
        if gl - off > 128 and n % 8 != 0:
            n -= n % 8
        pieces.append((off, n))
        off += n
    dcov = (dd + LANE - 1) // LANE * LANE
    offs = list(range(0, dcov, LANE))
    mesh = plsc.VectorSubcoreMesh(
        core_axis_name="c", subcore_axis_name="s", num_cores=NC, num_subcores=NS
    )

    @functools.partial(
        pl.kernel,
        out_type=jax.ShapeDtypeStruct((bb * dd,), jnp.float32),
        mesh=mesh,
        scratch_types=[
            pltpu.VMEM((tok_w,), jnp.int32),
            pltpu.VMEM((gl, dp), jnp.float32),
            pltpu.VMEM((g * dd + LANE,), jnp.float32),
            pltpu.SemaphoreType.DMA,
        ],
    )
    def k(x_hbm, tab_hbm, out_hbm, idx_v, rows_v, out_v, sem):
        w = lax.axis_index("s") * NC + lax.axis_index("c")
        pltpu.sync_copy(x_hbm.at[pl.ds(w * tok_w, tok_w)], idx_v)
        inv_l = jnp.float32(1.0 / ll)

        @pl.loop(0, nchunk)
        def _(c):
            base = c * gl
            cps = [
                pltpu.async_copy(
                    tab_hbm.at[idx_v.at[pl.ds(base + o, n)]],
                    rows_v.at[pl.ds(o, n)],
                    sem,
                )
                for o, n in pieces
            ]
            for cp in cps:
                cp.wait()
            for r in range(g):
                def body(l, acc):
                    t = r * ll + l
                    return tuple(
                        acc[i] + rows_v[t, pl.ds(o, LANE)]
                        for i, o in enumerate(offs)
                    )
                acc0 = tuple(jnp.zeros((LANE,), jnp.float32) for _ in offs)
                acc = lax.fori_loop(0, ll, body, acc0)
                for i, o in enumerate(offs):
                    out_v[pl.ds(r * dd + o, LANE)] = acc[i] * inv_l
            pltpu.sync_copy(
                out_v.at[pl.ds(0, g * dd)],
                out_hbm.at[pl.ds((w * nchunk + c) * g * dd, g * dd)],
            )

    return k(xi, table_p)


def _pad_table(table, dp):
    vv, dd = table.shape
    bs = 4000
    assert vv % bs == 0

    def body(x_ref, o_ref):
        o_ref[:, :dd] = x_ref[...]
        o_ref[:, dd:] = jnp.zeros((bs, dp - dd), jnp.float32)

    return pl.pallas_call(
        body,
        grid=(vv // bs,),
        in_specs=[pl.BlockSpec((bs, dd), lambda i: (i, 0))],
        out_specs=pl.BlockSpec((bs, dp), lambda i: (i, 0)),
        out_shape=jax.ShapeDtypeStruct((vv, dp), jnp.float32),
        compiler_params=pltpu.CompilerParams(
            dimension_semantics=("parallel",)
        ),
    )(table)


def kernel(x, table):
    bb, ll = x.shape
    _, dd = table.shape
    dp = (dd + 127) // 128 * 128
    xi = x.reshape(-1).astype(jnp.int32)
    table_p = _pad_table(table, dp)
    out = _emb_mean(xi, table_p, bb, ll, dd, dp, 4)
    return out.reshape(bb, dd)


# speedup vs baseline: 2.9906x; 1.1287x over previous
import functools

import jax
import jax.numpy as jnp
from jax import lax
from jax.experimental import pallas as pl
from jax.experimental.pallas import tpu as pltpu
from jax.experimental.pallas import tpu_sc as plsc

NC = 2
NS = 16
NW = NC * NS
LANE = 16


@functools.partial(jax.jit, static_argnames=("bb", "ll", "dd", "dp", "g"))
def _emb_mean(xi, table_p, bb, ll, dd, dp, g):
    tok_w = (bb // NW) * ll
    nchunk = (bb // NW) // g
    gl = g * ll
    pieces = []
    off = 0
    while off < gl:
        n = min(128, gl - off)
        if gl - off > 128 and n % 8 != 0:
            n -= n % 8
        pieces.append((off, n))
        off += n
    dcov = (dd + LANE - 1) // LANE * LANE
    offs = list(range(0, dcov, LANE))
    mesh = plsc.VectorSubcoreMesh(
        core_axis_name="c", subcore_axis_name="s", num_cores=NC, num_subcores=NS
    )

    h0, h1 = 104, gl - 104
    nacc = len(offs)

    @functools.partial(
        pl.kernel,
        out_type=jax.ShapeDtypeStruct((bb * dd,), jnp.float32),
        mesh=mesh,
        scratch_types=[
            pltpu.VMEM((tok_w,), jnp.int32),
            pltpu.VMEM((h0, dp), jnp.float32),
            pltpu.VMEM((h0, dp), jnp.float32),
            pltpu.VMEM((g * dd + LANE,), jnp.float32),
            pltpu.VMEM((g * dd + LANE,), jnp.float32),
            pltpu.VMEM((dcov,), jnp.float32),
            pltpu.SemaphoreType.DMA,
            pltpu.SemaphoreType.DMA,
            pltpu.SemaphoreType.DMA,
            pltpu.SemaphoreType.DMA,
        ],
    )
    def k(x_hbm, tab_hbm, out_hbm, idx_v, rows0_v, rows1_v, outb0_v, outb1_v,
          carry_v, sg0, sg1, so0, so1):
        rows_refs = (rows0_v, rows1_v)
        outb_refs = (outb0_v, outb1_v)
        w = lax.axis_index("s") * NC + lax.axis_index("c")
        pltpu.sync_copy(x_hbm.at[pl.ds(w * tok_w, tok_w)], idx_v)
        inv_l = jnp.float32(1.0 / ll)
        sgs = (sg0, sg1)
        sos = (so0, so1)
        ow = w * nchunk * g * dd

        def issue_gather(c, half):
            if half == 0:
                pltpu.async_copy(
                    tab_hbm.at[idx_v.at[pl.ds(c * gl, h0)]],
                    rows0_v,
                    sgs[0],
                )
            else:
                pltpu.async_copy(
                    tab_hbm.at[idx_v.at[pl.ds(c * gl + h0, h1)]],
                    rows1_v.at[pl.ds(0, h1)],
                    sgs[1],
                )

        def wait_gather(half):
            if half == 0:
                pltpu.make_async_copy(
                    tab_hbm.at[pl.ds(0, h0)], rows0_v, sgs[0]
                ).wait()
            else:
                pltpu.make_async_copy(
                    tab_hbm.at[pl.ds(0, h1)],
                    rows1_v.at[pl.ds(0, h1)],
                    sgs[1],
                ).wait()

        def reduce_span(slot, t0, n, carry_in):
            if carry_in:
                acc0 = tuple(
                    carry_v[pl.ds(o, LANE)] for o in offs
                )
            else:
                acc0 = tuple(jnp.zeros((LANE,), jnp.float32) for _ in offs)

            def body(l, acc):
                return tuple(
                    acc[i] + rows_refs[slot][t0 + l, pl.ds(o, LANE)]
                    for i, o in enumerate(offs)
                )

            return lax.fori_loop(0, n, body, acc0)

        def store_row(cb, r, acc):
            for i, o in enumerate(offs):
                outb_refs[cb][pl.ds(r * dd + o, LANE)] = acc[i] * inv_l

        for b in (0, 1):
            pltpu.async_copy(
                out_hbm.at[pl.ds(0, g * dd)],
                outb_refs[b].at[pl.ds(0, g * dd)],
                sos[b],
            )
        issue_gather(0, 0)
        issue_gather(0, 1)

        @pl.loop(0, nchunk, step=2)
        def _(cbase):
            for cb in (0, 1):
                c = cbase + cb
                wait_gather(0)
                pltpu.make_async_copy(
                    out_hbm.at[pl.ds(0, g * dd)],
                    outb_refs[cb].at[pl.ds(0, g * dd)],
                    sos[cb],
                ).wait()
                store_row(cb, 0, reduce_span(0, 0, ll, False))
                store_row(cb, 1, reduce_span(0, ll, ll, False))
                acc = reduce_span(0, 2 * ll, h0 - 2 * ll, False)
                for i, o in enumerate(offs):
                    carry_v[pl.ds(o, LANE)] = acc[i]

                @pl.when(c + 1 < nchunk)
                def _():
                    issue_gather(c + 1, 0)

                wait_gather(1)
                n2 = 3 * ll - h0
                store_row(cb, 2, reduce_span(1, 0, n2, True))
                store_row(cb, 3, reduce_span(1, n2, ll, False))
                pltpu.async_copy(
                    outb_refs[cb].at[pl.ds(0, g * dd)],
                    out_hbm.at[pl.ds(ow + c * g * dd, g * dd)],
                    sos[cb],
                )

                @pl.when(c + 1 < nchunk)
                def _():
                    issue_gather(c + 1, 1)

        for b in (0, 1):
            pltpu.make_async_copy(
                out_hbm.at[pl.ds(0, g * dd)],
                outb_refs[b].at[pl.ds(0, g * dd)],
                sos[b],
            ).wait()

    return k(xi, table_p)


def _pad_table(table, dp):
    vv, dd = table.shape
    bs = 4000
    assert vv % bs == 0

    def body(x_ref, o_ref):
        o_ref[:, :dd] = x_ref[...]
        o_ref[:, dd:] = jnp.zeros((bs, dp - dd), jnp.float32)

    return pl.pallas_call(
        body,
        grid=(vv // bs,),
        in_specs=[pl.BlockSpec((bs, dd), lambda i: (i, 0))],
        out_specs=pl.BlockSpec((bs, dp), lambda i: (i, 0)),
        out_shape=jax.ShapeDtypeStruct((vv, dp), jnp.float32),
        compiler_params=pltpu.CompilerParams(
            dimension_semantics=("parallel",)
        ),
    )(table)


def kernel(x, table):
    bb, ll = x.shape
    _, dd = table.shape
    dp = (dd + 127) // 128 * 128
    xi = x.reshape(-1).astype(jnp.int32)
    table_p = _pad_table(table, dp)
    out = _emb_mean(xi, table_p, bb, ll, dd, dp, 4)
    return out.reshape(bb, dd)


# speedup vs baseline: 3.0025x; 1.0040x over previous
import functools

import jax
import jax.numpy as jnp
from jax import lax
from jax.experimental import pallas as pl
from jax.experimental.pallas import tpu as pltpu
from jax.experimental.pallas import tpu_sc as plsc

NC = 2
NS = 16
NW = NC * NS
LANE = 16


@functools.partial(jax.jit, static_argnames=("bb", "ll", "dd", "dp", "g"))
def _emb_mean(xi, table_p, bb, ll, dd, dp, g):
    tok_w = (bb // NW) * ll
    nchunk = (bb // NW) // g
    gl = g * ll
    pieces = []
    off = 0
    while off < gl:
        n = min(128, gl - off)
        if gl - off > 128 and n % 8 != 0:
            n -= n % 8
        pieces.append((off, n))
        off += n
    dcov = (dd + LANE - 1) // LANE * LANE
    offs = list(range(0, dcov, LANE))
    mesh = plsc.VectorSubcoreMesh(
        core_axis_name="c", subcore_axis_name="s", num_cores=NC, num_subcores=NS
    )

    h0, h1 = 104, gl - 104
    nacc = len(offs)

    @functools.partial(
        pl.kernel,
        out_type=jax.ShapeDtypeStruct((bb * dd,), jnp.float32),
        mesh=mesh,
        scratch_types=[
            pltpu.VMEM((tok_w,), jnp.int32),
            pltpu.VMEM((h0, dp), jnp.float32),
            pltpu.VMEM((h0, dp), jnp.float32),
            pltpu.VMEM((g * dd + LANE,), jnp.float32),
            pltpu.VMEM((g * dd + LANE,), jnp.float32),
            pltpu.VMEM((dcov,), jnp.float32),
            pltpu.SemaphoreType.DMA,
            pltpu.SemaphoreType.DMA,
            pltpu.SemaphoreType.DMA,
            pltpu.SemaphoreType.DMA,
        ],
    )
    def k(x_hbm, tab_hbm, out_hbm, idx_v, rows0_v, rows1_v, outb0_v, outb1_v,
          carry_v, sg0, sg1, so0, so1):
        rows_refs = (rows0_v, rows1_v)
        outb_refs = (outb0_v, outb1_v)
        w = lax.axis_index("s") * NC + lax.axis_index("c")
        pltpu.sync_copy(x_hbm.at[pl.ds(w * tok_w, tok_w)], idx_v)
        inv_l = jnp.float32(1.0 / ll)
        sgs = (sg0, sg1)
        sos = (so0, so1)
        ow = w * nchunk * g * dd

        def issue_gather(c, half):
            if half == 0:
                pltpu.async_copy(
                    tab_hbm.at[idx_v.at[pl.ds(c * gl, h0)]],
                    rows0_v,
                    sgs[0],
                )
            else:
                pltpu.async_copy(
                    tab_hbm.at[idx_v.at[pl.ds(c * gl + h0, h1)]],
                    rows1_v.at[pl.ds(0, h1)],
                    sgs[1],
                )

        def wait_gather(half):
            if half == 0:
                pltpu.make_async_copy(
                    tab_hbm.at[pl.ds(0, h0)], rows0_v, sgs[0]
                ).wait()
            else:
                pltpu.make_async_copy(
                    tab_hbm.at[pl.ds(0, h1)],
                    rows1_v.at[pl.ds(0, h1)],
                    sgs[1],
                ).wait()

        def reduce_span(slot, t0, n, carry_in):
            if carry_in:
                acc0 = tuple(
                    carry_v[pl.ds(o, LANE)] for o in offs
                )
            else:
                acc0 = tuple(jnp.zeros((LANE,), jnp.float32) for _ in offs)

            def body(l, acc):
                return tuple(
                    acc[i] + rows_refs[slot][t0 + l, pl.ds(o, LANE)]
                    for i, o in enumerate(offs)
                )

            return lax.fori_loop(0, n, body, acc0)

        def store_row(cb, r, acc):
            for i, o in enumerate(offs):
                outb_refs[cb][pl.ds(r * dd + o, LANE)] = acc[i] * inv_l

        for b in (0, 1):
            pltpu.async_copy(
                out_hbm.at[pl.ds(0, g * dd)],
                outb_refs[b].at[pl.ds(0, g * dd)],
                sos[b],
            )
        issue_gather(0, 0)
        issue_gather(0, 1)

        @pl.loop(0, nchunk, step=2)
        def _(cbase):
            for cb in (0, 1):
                c = cbase + cb
                wait_gather(0)
                pltpu.make_async_copy(
                    out_hbm.at[pl.ds(0, g * dd)],
                    outb_refs[cb].at[pl.ds(0, g * dd)],
                    sos[cb],
                ).wait()
                store_row(cb, 0, reduce_span(0, 0, ll, False))
                store_row(cb, 1, reduce_span(0, ll, ll, False))
                acc = reduce_span(0, 2 * ll, h0 - 2 * ll, False)
                for i, o in enumerate(offs):
                    carry_v[pl.ds(o, LANE)] = acc[i]

                @pl.when(c + 1 < nchunk)
                def _():
                    issue_gather(c + 1, 0)

                wait_gather(1)
                n2 = 3 * ll - h0
                store_row(cb, 2, reduce_span(1, 0, n2, True))
                store_row(cb, 3, reduce_span(1, n2, ll, False))
                pltpu.async_copy(
                    outb_refs[cb].at[pl.ds(0, g * dd)],
                    out_hbm.at[pl.ds(ow + c * g * dd, g * dd)],
                    sos[cb],
                )

                @pl.when(c + 1 < nchunk)
                def _():
                    issue_gather(c + 1, 1)

        for b in (0, 1):
            pltpu.make_async_copy(
                out_hbm.at[pl.ds(0, g * dd)],
                outb_refs[b].at[pl.ds(0, g * dd)],
                sos[b],
            ).wait()

    return k(xi, table_p)


def _pad_table(table, dp):
    vv, dd = table.shape
    bs = 10000
    assert vv % bs == 0

    def body(x_ref, o_ref):
        o_ref[:, :dd] = x_ref[...]
        o_ref[:, dd:] = jnp.zeros((bs, dp - dd), jnp.float32)

    return pl.pallas_call(
        body,
        grid=(vv // bs,),
        in_specs=[pl.BlockSpec((bs, dd), lambda i: (i, 0))],
        out_specs=pl.BlockSpec((bs, dp), lambda i: (i, 0)),
        out_shape=jax.ShapeDtypeStruct((vv, dp), jnp.float32),
        compiler_params=pltpu.CompilerParams(
            dimension_semantics=("parallel",)
        ),
    )(table)


def kernel(x, table):
    bb, ll = x.shape
    _, dd = table.shape
    dp = (dd + 127) // 128 * 128
    xi = x.reshape(-1).astype(jnp.int32)
    table_p = _pad_table(table, dp)
    out = _emb_mean(xi, table_p, bb, ll, dd, dp, 4)
    return out.reshape(bb, dd)
